# baseline (speedup 1.0000x reference)
import functools

import jax
import jax.numpy as jnp
from jax import lax
from jax.experimental import pallas as pl
from jax.experimental.pallas import tpu as pltpu
from jax.experimental.pallas import tpu_sc as plsc

N_BINS = 1024
HIDDEN = 16
MIN_VAL = -3.0
MAX_VAL = 3.0
BATCH = 16384
FIELDS = 100
NC, NS, LANES = 2, 16, 16
NW = NC * NS
BW = BATCH // NW
BH = 256
FB = 10
HT, HS, BT, BL = HIDDEN // 8, 8, BATCH // 128, 128
NPAIR = FIELDS // (2 * FB)
INV_STEP = float(N_BINS - 2) / (MAX_VAL - MIN_VAL)


@functools.cache
def _build():
    mesh = plsc.VectorSubcoreMesh(core_axis_name="c", subcore_axis_name="s")

    @functools.partial(
        pl.kernel,
        mesh=mesh,
        out_type=jax.ShapeDtypeStruct((FIELDS, HT, BT, HS, BL), jnp.float32),
        scratch_types=[
            pltpu.VMEM((N_BINS - 1,), jnp.float32),
            pltpu.VMEM((N_BINS * (HIDDEN + 1),), jnp.float32),
            pltpu.VMEM((FIELDS, BH), jnp.float32),
            pltpu.VMEM((FB, HT, BH // 128, HS, BL), jnp.float32),
            pltpu.VMEM((FB, HT, BH // 128, HS, BL), jnp.float32),
            pltpu.SemaphoreType.DMA,
            pltpu.SemaphoreType.DMA,
        ],
        compiler_params=pltpu.CompilerParams(
            use_tc_tiling_on_sc=False, needs_layout_passes=False
        ),
    )
    def _sc_embed(xt_hbm, bounds_hbm, tflat_hbm, out_hbm,
                  bounds_v, tv, xv, vb0, vb1, sem0, sem1):
        wid = lax.axis_index("s") * NC + lax.axis_index("c")
        pltpu.sync_copy(bounds_hbm, bounds_v)
        pltpu.sync_copy(tflat_hbm, tv)
        iota = lax.iota(jnp.int32, LANES)

        def half(ph, carry):
            b0 = wid * BW + ph * BH
            bt0 = b0 // 128
            pltpu.sync_copy(xt_hbm.at[:, pl.ds(b0, BH)], xv)

            def fields_into(vb, f0base):
                @plsc.parallel_loop(0, FB)
                def field(f_):
                    fa = f0base + f_

                    @plsc.parallel_loop(0, BH // LANES, unroll=8)
                    def bwin(w):
                        xx = xv[fa, pl.ds(w * LANES, LANES)]
                        t = jnp.clip((xx - MIN_VAL) * INV_STEP,
                                     0.0, float(N_BINS - 1))
                        g = t.astype(jnp.int32)
                        g = g + jnp.where(t > g.astype(jnp.float32), 1, 0)
                        bu = plsc.load_gather(
                            bounds_v, [jnp.minimum(g, N_BINS - 2)])
                        bd = plsc.load_gather(
                            bounds_v, [jnp.maximum(g - 1, 0)])
                        up = jnp.where((g < N_BINS - 1) & (bu < xx), 1, 0)
                        dn = jnp.where((g > 0) & (bd >= xx), 1, 0)
                        av = (g + up - dn) * (HIDDEN + 1)
                        bt_ = w // 8
                        bl_ = (w % 8) * LANES
                        for h in range(HIDDEN):
                            vals = plsc.load_gather(tv, [av + h])
                            vb[f_, h // 8, bt_, h % 8, pl.ds(bl_, LANES)] = vals

            def pair(p, c):
                f0 = p * 2 * FB

                @pl.when(p > 0)
                def _():
                    pltpu.make_async_copy(
                        vb0, out_hbm.at[pl.ds(0, FB), :, pl.ds(bt0, BH // 128)],
                        sem0).wait()

                fields_into(vb0, f0)
                pltpu.async_copy(
                    vb0, out_hbm.at[pl.ds(f0, FB), :, pl.ds(bt0, BH // 128)],
                    sem0)

                @pl.when(p > 0)
                def _():
                    pltpu.make_async_copy(
                        vb1, out_hbm.at[pl.ds(0, FB), :, pl.ds(bt0, BH // 128)],
                        sem1).wait()

                fields_into(vb1, f0 + FB)
                pltpu.async_copy(
                    vb1, out_hbm.at[pl.ds(f0 + FB, FB), :,
                                    pl.ds(bt0, BH // 128)],
                    sem1)
                return c

            lax.fori_loop(0, NPAIR, pair, 0, unroll=False)
            pltpu.make_async_copy(
                vb0, out_hbm.at[pl.ds(0, FB), :, pl.ds(bt0, BH // 128)],
                sem0).wait()
            pltpu.make_async_copy(
                vb1, out_hbm.at[pl.ds(0, FB), :, pl.ds(bt0, BH // 128)],
                sem1).wait()
            return carry

        lax.fori_loop(0, BW // BH, half, 0, unroll=False)

    return _sc_embed


def kernel(x, bounds, table):
    tpad = jnp.pad(table, ((0, 0), (0, 1)))
    raw = _build()(x.T, bounds, tpad.reshape(N_BINS * (HIDDEN + 1)))
    out = raw.transpose(2, 4, 0, 1, 3)
    return out.reshape(BATCH, FIELDS, HIDDEN)


# speedup vs baseline: 1.1290x; 1.1290x over previous
import functools

import jax
import jax.numpy as jnp
from jax import lax
from jax.experimental import pallas as pl
from jax.experimental.pallas import tpu as pltpu
from jax.experimental.pallas import tpu_sc as plsc

N_BINS = 1024
HIDDEN = 16
MIN_VAL = -3.0
MAX_VAL = 3.0
BATCH = 16384
FIELDS = 100
NC, NS, LANES = 2, 16, 16
NW = NC * NS
BW = BATCH // NW
BH = 256
FB = 10
HT, HS, BT, BL = HIDDEN // 8, 8, BATCH // 128, 128
NPAIR = FIELDS // (2 * FB)
INV_STEP = float(N_BINS - 2) / (MAX_VAL - MIN_VAL)


@functools.cache
def _build():
    mesh = plsc.VectorSubcoreMesh(core_axis_name="c", subcore_axis_name="s")

    @functools.partial(
        pl.kernel,
        mesh=mesh,
        out_type=jax.ShapeDtypeStruct((FIELDS, HT, BT, HS, BL), jnp.float32),
        scratch_types=[
            pltpu.VMEM((N_BINS - 1,), jnp.float32),
            pltpu.VMEM((N_BINS * (HIDDEN + 1),), jnp.float32),
            pltpu.VMEM((FIELDS, BH), jnp.float32),
            pltpu.VMEM((FB, HT, BH // 128, HS, BL), jnp.float32),
            pltpu.VMEM((FB, HT, BH // 128, HS, BL), jnp.float32),
            pltpu.SemaphoreType.DMA,
            pltpu.SemaphoreType.DMA,
        ],
        compiler_params=pltpu.CompilerParams(
            use_tc_tiling_on_sc=False, needs_layout_passes=False
        ),
    )
    def _sc_embed(xt_hbm, bounds_hbm, tflat_hbm, out_hbm,
                  bounds_v, tv, xv, vb0, vb1, sem0, sem1):
        wid = lax.axis_index("s") * NC + lax.axis_index("c")
        pltpu.sync_copy(bounds_hbm, bounds_v)
        pltpu.sync_copy(tflat_hbm, tv)
        iota = lax.iota(jnp.int32, LANES)

        def half(ph, carry):
            b0 = wid * BW + ph * BH
            bt0 = b0 // 128
            pltpu.sync_copy(xt_hbm.at[:, pl.ds(b0, BH)], xv)

            def fields_into(vb, f0base):
                @plsc.parallel_loop(0, FB)
                def field(f_):
                    fa = f0base + f_

                    @plsc.parallel_loop(0, BH // LANES, unroll=2)
                    def bwin(w):
                        xx = xv[fa, pl.ds(w * LANES, LANES)]
                        t = jnp.clip((xx - MIN_VAL) * INV_STEP,
                                     0.0, float(N_BINS - 1))
                        g = t.astype(jnp.int32)
                        g = g + jnp.where(t > g.astype(jnp.float32), 1, 0)
                        bu = plsc.load_gather(
                            bounds_v, [jnp.minimum(g, N_BINS - 2)])
                        bd = plsc.load_gather(
                            bounds_v, [jnp.maximum(g - 1, 0)])
                        up = jnp.where((g < N_BINS - 1) & (bu < xx), 1, 0)
                        dn = jnp.where((g > 0) & (bd >= xx), 1, 0)
                        av = (g + up - dn) * (HIDDEN + 1)
                        bt_ = w // 8
                        bl_ = (w % 8) * LANES
                        for h in range(HIDDEN):
                            vals = plsc.load_gather(tv, [av + h])
                            vb[f_, h // 8, bt_, h % 8, pl.ds(bl_, LANES)] = vals

            def pair(p, c):
                f0 = p * 2 * FB

                @pl.when(p > 0)
                def _():
                    pltpu.make_async_copy(
                        vb0, out_hbm.at[pl.ds(0, FB), :, pl.ds(bt0, BH // 128)],
                        sem0).wait()

                fields_into(vb0, f0)
                pltpu.async_copy(
                    vb0, out_hbm.at[pl.ds(f0, FB), :, pl.ds(bt0, BH // 128)],
                    sem0)

                @pl.when(p > 0)
                def _():
                    pltpu.make_async_copy(
                        vb1, out_hbm.at[pl.ds(0, FB), :, pl.ds(bt0, BH // 128)],
                        sem1).wait()

                fields_into(vb1, f0 + FB)
                pltpu.async_copy(
                    vb1, out_hbm.at[pl.ds(f0 + FB, FB), :,
                                    pl.ds(bt0, BH // 128)],
                    sem1)
                return c

            lax.fori_loop(0, NPAIR, pair, 0, unroll=False)
            pltpu.make_async_copy(
                vb0, out_hbm.at[pl.ds(0, FB), :, pl.ds(bt0, BH // 128)],
                sem0).wait()
            pltpu.make_async_copy(
                vb1, out_hbm.at[pl.ds(0, FB), :, pl.ds(bt0, BH // 128)],
                sem1).wait()
            return carry

        lax.fori_loop(0, BW // BH, half, 0, unroll=False)

    return _sc_embed


def kernel(x, bounds, table):
    tpad = jnp.pad(table, ((0, 0), (0, 1)))
    raw = _build()(x.T, bounds, tpad.reshape(N_BINS * (HIDDEN + 1)))
    out = raw.transpose(2, 4, 0, 1, 3)
    return out.reshape(BATCH, FIELDS, HIDDEN)


# speedup vs baseline: 1.1295x; 1.0004x over previous
import functools

import jax
import jax.numpy as jnp
from jax import lax
from jax.experimental import pallas as pl
from jax.experimental.pallas import tpu as pltpu
from jax.experimental.pallas import tpu_sc as plsc

N_BINS = 1024
HIDDEN = 16
MIN_VAL = -3.0
MAX_VAL = 3.0
BATCH = 16384
FIELDS = 100
NC, NS, LANES = 2, 16, 16
NW = NC * NS
BW = BATCH // NW
BH = 256
FB = 10
HT, HS, BT, BL = HIDDEN // 8, 8, BATCH // 128, 128
NPAIR = FIELDS // (2 * FB)
INV_STEP = float(N_BINS - 2) / (MAX_VAL - MIN_VAL)


@functools.cache
def _build():
    mesh = plsc.VectorSubcoreMesh(core_axis_name="c", subcore_axis_name="s")

    @functools.partial(
        pl.kernel,
        mesh=mesh,
        out_type=jax.ShapeDtypeStruct((FIELDS, HT, BT, HS, BL), jnp.float32),
        scratch_types=[
            pltpu.VMEM((N_BINS - 1,), jnp.float32),
            pltpu.VMEM((N_BINS * (HIDDEN + 1),), jnp.float32),
            pltpu.VMEM((FIELDS, BH), jnp.float32),
            pltpu.VMEM((FB, HT, BH // 128, HS, BL), jnp.float32),
            pltpu.VMEM((FB, HT, BH // 128, HS, BL), jnp.float32),
            pltpu.SemaphoreType.DMA,
            pltpu.SemaphoreType.DMA,
        ],
        compiler_params=pltpu.CompilerParams(
            use_tc_tiling_on_sc=False, needs_layout_passes=False
        ),
    )
    def _sc_embed(xt_hbm, bounds_hbm, tflat_hbm, out_hbm,
                  bounds_v, tv, xv, vb0, vb1, sem0, sem1):
        wid = lax.axis_index("s") * NC + lax.axis_index("c")
        pltpu.sync_copy(bounds_hbm, bounds_v)
        pltpu.sync_copy(tflat_hbm, tv)

        def half(ph, carry):
            b0 = wid * BW + ph * BH
            bt0 = b0 // 128
            pltpu.sync_copy(xt_hbm.at[:, pl.ds(b0, BH)], xv)

            def fields_into(vb, f0base):
                @plsc.parallel_loop(0, FB)
                def field(f_):
                    fa = f0base + f_

                    @plsc.parallel_loop(0, BH // LANES, unroll=4)
                    def bwin(w):
                        xx = xv[fa, pl.ds(w * LANES, LANES)]
                        t = jnp.clip((xx - MIN_VAL) * INV_STEP,
                                     0.0, float(N_BINS - 1))
                        g = t.astype(jnp.int32)
                        g = g + jnp.where(t > g.astype(jnp.float32), 1, 0)
                        bu = plsc.load_gather(
                            bounds_v, [jnp.minimum(g, N_BINS - 2)])
                        bd = plsc.load_gather(
                            bounds_v, [jnp.maximum(g - 1, 0)])
                        up = jnp.where((g < N_BINS - 1) & (bu < xx), 1, 0)
                        dn = jnp.where((g > 0) & (bd >= xx), 1, 0)
                        av = (g + up - dn) * (HIDDEN + 1)
                        bt_ = w // 8
                        bl_ = (w % 8) * LANES
                        for h in range(HIDDEN):
                            vals = plsc.load_gather(tv, [av + h])
                            vb[f_, h // 8, bt_, h % 8, pl.ds(bl_, LANES)] = vals

            def pair(p, c):
                f0 = p * 2 * FB

                @pl.when(p > 0)
                def _():
                    pltpu.make_async_copy(
                        vb0, out_hbm.at[pl.ds(0, FB), :, pl.ds(bt0, BH // 128)],
                        sem0).wait()

                fields_into(vb0, f0)
                pltpu.async_copy(
                    vb0, out_hbm.at[pl.ds(f0, FB), :, pl.ds(bt0, BH // 128)],
                    sem0)

                @pl.when(p > 0)
                def _():
                    pltpu.make_async_copy(
                        vb1, out_hbm.at[pl.ds(0, FB), :, pl.ds(bt0, BH // 128)],
                        sem1).wait()

                fields_into(vb1, f0 + FB)
                pltpu.async_copy(
                    vb1, out_hbm.at[pl.ds(f0 + FB, FB), :,
                                    pl.ds(bt0, BH // 128)],
                    sem1)
                return c

            lax.fori_loop(0, NPAIR, pair, 0, unroll=False)
            pltpu.make_async_copy(
                vb0, out_hbm.at[pl.ds(0, FB), :, pl.ds(bt0, BH // 128)],
                sem0).wait()
            pltpu.make_async_copy(
                vb1, out_hbm.at[pl.ds(0, FB), :, pl.ds(bt0, BH // 128)],
                sem1).wait()
            return carry

        lax.fori_loop(0, BW // BH, half, 0, unroll=False)

    return _sc_embed


def kernel(x, bounds, table):
    tpad = jnp.pad(table, ((0, 0), (0, 1)))
    raw = _build()(x.T, bounds, tpad.reshape(N_BINS * (HIDDEN + 1)))
    out = raw.transpose(2, 4, 0, 1, 3)
    return out.reshape(BATCH, FIELDS, HIDDEN)
